# baseline (speedup 1.0000x reference)
import functools

import jax
import jax.numpy as jnp
from jax import lax
from jax.experimental import pallas as pl
from jax.experimental.pallas import tpu as pltpu
from jax.experimental.pallas import tpu_sc as plsc


_U = 10000
_I = 5000
_G = 2000
_D = 64
_LG_ITEM = 3000
_N_LG = _G + _LG_ITEM
_B = 16384
_F32 = jnp.float32
_BF16 = jnp.bfloat16

_BM = 200
_FH_BM = 200
_S_C = 0
_S_D = 10
_S_E = 120
_S_F = 130
_S_G = 155
_STEPS = 156


def _bdot(a, b):
    return jnp.dot(a.astype(_BF16), b.astype(_BF16),
                   preferred_element_type=_F32)


def _overlap_body(*args):
    a_parts, (g_ref, out_ref, c1_ref) = args[:_OV_SPLIT], args[_OV_SPLIT:]
    q = _G // _OV_SPLIT
    parts = a_parts
    g = g_ref[...]
    for k, a_ref in enumerate(parts):
        c1_ref[pl.ds(k * q, q), :] = _bdot(a_ref[...], g)
    c1 = c1_ref[...]
    for k, a_ref in enumerate(parts):
        sl = pl.ds(k * q, q)
        out_ref[sl, :] = g_ref[sl, :] + c1_ref[sl, :] + _bdot(a_ref[...], c1)


_OV_SPLIT = 5


def _overlap_conv(overlap_graph, group_table):
    q = _G // _OV_SPLIT
    return pl.pallas_call(
        _overlap_body,
        grid=(1,),
        in_specs=[pl.BlockSpec((q, _G), lambda i, k=k: (k, 0))
                  for k in range(_OV_SPLIT)]
        + [pl.BlockSpec((_G, _D), lambda i: (0, 0))],
        out_specs=pl.BlockSpec((_G, _D), lambda i: (0, 0)),
        out_shape=jax.ShapeDtypeStruct((_G, _D), _F32),
        scratch_shapes=[pltpu.VMEM((_G, _D), _F32)],
    )(*([overlap_graph] * _OV_SPLIT + [group_table]))


def _mega_body(lg_ref, uh_ref, ih_ref, fh_ref, u_ref, it_ref, g_ref,
               ge_ref, w_ref, b_ref, wov_ref, bov_ref, why_ref, bhy_ref,
               wlg_ref, blg_ref, iemb_ref, gui_ref,
               e0_s, cur1_s, lgem_s, msg0_s, msg1_s, norm0_s):
    i = pl.program_id(0)
    ge_s = ge_ref

    @pl.when(i == 0)
    def _():
        e0_s[0:_G, :] = g_ref[...].astype(_BF16)
        e0_s[_G:, :] = it_ref[0:_LG_ITEM, :].astype(_BF16)

    def _msg(dst_s, m, u_st, it_st):
        um = _bdot(uh_ref[...], u_st)
        im = _bdot(ih_ref[...], it_st)
        ige = im * ge_s[pl.ds(m * _BM, _BM), :]
        w = w_ref[0]
        dst_s[pl.ds(m * _BM, _BM), :] = (
            jnp.dot(um, w[0:_D], preferred_element_type=_F32)
            + jnp.dot(im, w[_D:2 * _D], preferred_element_type=_F32)
            + jnp.dot(ige, w[2 * _D:3 * _D], preferred_element_type=_F32)
            + b_ref[0])

    @pl.when(i < _S_D)
    def _():
        _msg(msg0_s, i - _S_C, u_ref[...], it_ref[...])

    @pl.when((i >= _S_D) & (i < _S_E)
             & (((i - _S_D) % 2 == 0) | (i >= _S_D + 70)))
    def _():
        k = jnp.where(i < _S_D + 70, (i - _S_D) // 2, 35 + (i - (_S_D + 70)))
        norm0_s[pl.ds(k * _FH_BM, _FH_BM), :] = _bdot(fh_ref[...],
                                                      msg0_s[...])

    @pl.when((i >= _S_D) & (i < _S_D + 70) & ((i - _S_D) % 2 == 1))
    def _():
        slot = jnp.clip((i - (_S_D + 1)) // 2, 0, 34)

        @pl.when(slot < 25)
        def _():
            cur1_s[pl.ds(slot * _BM, _BM), :] = _bdot(lg_ref[...],
                                                      e0_s[...])

        @pl.when(slot >= 25)
        def _():
            j2 = slot - 25
            sl = pl.ds(j2 * _BM, _BM)
            lgem_s[sl, :] = ((g_ref[sl, :] + cur1_s[sl, :]
                              + _bdot(lg_ref[...], cur1_s[...]))
                             * (1.0 / 3.0))

    @pl.when((i >= _S_E) & (i < _S_F))
    def _():
        _msg(msg1_s, i - _S_E, norm0_s[0:_U, :], norm0_s[_U:, :])

    @pl.when((i >= _S_F) & (i < _S_G))
    def _():
        k = i - _S_F
        iemb_ref[...] = (it_ref[pl.ds(k * _FH_BM, _FH_BM), :]
                         + norm0_s[pl.ds(_U + k * _FH_BM, _FH_BM), :]
                         + _bdot(fh_ref[...], msg1_s[...]))

    @pl.when(i == _S_G)
    def _():
        ge = ge_s[...]
        he = ge + msg0_s[...] + msg1_s[...]
        lg = lgem_s[...]
        co = jax.nn.sigmoid(jnp.dot(ge, wov_ref[...],
                                    preferred_element_type=_F32) + bov_ref[...])
        ch = jax.nn.sigmoid(jnp.dot(he, why_ref[...],
                                    preferred_element_type=_F32) + bhy_ref[...])
        cl = jax.nn.sigmoid(jnp.dot(lg, wlg_ref[...],
                                    preferred_element_type=_F32) + blg_ref[...])
        gui_ref[...] = co * ge + ch * he + cl * lg


def _lg_idx(i):
    slot = jnp.clip((i - (_S_D + 1)) // 2, 0, 34)
    blk = jnp.where(slot < 25, slot, slot - 25)
    return (jnp.where(i < _S_D + 1, 0,
                      jnp.where(i < _S_D + 70, blk, 9)), 0)


def _uhih_idx(i):
    return (jnp.where(
        i < _S_C, 0,
        jnp.where(i < _S_D, i - _S_C,
                  jnp.where(i < _S_E, 9,
                            jnp.where(i < _S_F, i - _S_E, 9)))), 0)


def _fh_idx(i):
    d_blk = jnp.where(i < _S_D + 70, jnp.clip((i - _S_D) // 2, 0, 34),
                      35 + jnp.clip(i - (_S_D + 70), 0, 39))
    return (jnp.where(
        i < _S_D, 0,
        jnp.where(i < _S_E, d_blk,
                  jnp.where(i < _S_F, 74,
                            _U // _FH_BM + jnp.clip(i - _S_F, 0, 24)))), 0)


def _w_idx(i):
    return (jnp.where(i < _S_E, 0, 1), 0, 0)


def _fused_dense(user_table, item_table, group_table, group_emb, user_hyper,
                 item_hyper, full_hyper, lgcn_graph, W_agg, b_agg3,
                 W_ov, b_ov, W_hy, b_hy, W_lg, b_lg):
    const2 = lambda i: (0, 0)
    const1 = lambda i: (0,)
    return pl.pallas_call(
        _mega_body,
        grid=(_STEPS,),
        in_specs=[
            pl.BlockSpec((_BM, _N_LG), _lg_idx),
            pl.BlockSpec((_BM, _U), _uhih_idx),
            pl.BlockSpec((_BM, _I), _uhih_idx),
            pl.BlockSpec((_FH_BM, _G), _fh_idx),
            pl.BlockSpec((_U, _D), const2),
            pl.BlockSpec((_I, _D), const2),
            pl.BlockSpec((_G, _D), const2),
            pl.BlockSpec((_G, _D), const2),
            pl.BlockSpec((1, 3 * _D, _D), _w_idx),
            pl.BlockSpec((1, 1, _D), _w_idx),
            pl.BlockSpec((_D, 1), const2),
            pl.BlockSpec((1,), const1),
            pl.BlockSpec((_D, 1), const2),
            pl.BlockSpec((1,), const1),
            pl.BlockSpec((_D, 1), const2),
            pl.BlockSpec((1,), const1),
        ],
        out_specs=[
            pl.BlockSpec((_FH_BM, _D),
                         lambda i: (jnp.clip(i - _S_F, 0, 24), 0)),
            pl.BlockSpec((_G, _D), lambda i: (0, 0)),
        ],
        out_shape=[
            jax.ShapeDtypeStruct((_I, _D), _F32),
            jax.ShapeDtypeStruct((_G, _D), _F32),
        ],
        scratch_shapes=[pltpu.VMEM((_N_LG, _D), _BF16),
                        pltpu.VMEM((_N_LG, _D), _F32),
                        pltpu.VMEM((_G, _D), _F32),
                        pltpu.VMEM((_G, _D), _F32),
                        pltpu.VMEM((_G, _D), _F32),
                        pltpu.VMEM((_U + _I, _D), _F32)],
        compiler_params=pltpu.CompilerParams(
            vmem_limit_bytes=100 * 1024 * 1024),
    )(lgcn_graph, user_hyper, item_hyper, full_hyper,
      user_table, item_table, group_table, group_emb, W_agg, b_agg3,
      W_ov, b_ov, W_hy, b_hy, W_lg, b_lg)


def _dot_body(g_ref, i_ref, out_ref):
    out_ref[...] = jnp.sum(g_ref[...] * i_ref[...], axis=1)


def _pair_dot(g_sel, i_sel):
    bm = 4096
    grid = (_B // bm,)
    return pl.pallas_call(
        _dot_body,
        grid=grid,
        in_specs=[
            pl.BlockSpec((bm, _D), lambda i: (i, 0)),
            pl.BlockSpec((bm, _D), lambda i: (i, 0)),
        ],
        out_specs=pl.BlockSpec((bm,), lambda i: (i,)),
        out_shape=jax.ShapeDtypeStruct((_B,), _F32),
    )(g_sel, i_sel)


_NC = 2
_NS = 16
_NW = _NC * _NS
_BPW = _B // _NW


def _sc_gather_pair(g_tab, i_tab, g_idx, i_idx):
    mesh = plsc.VectorSubcoreMesh(core_axis_name="c", subcore_axis_name="s")

    @functools.partial(
        pl.kernel,
        mesh=mesh,
        out_type=[
            jax.ShapeDtypeStruct((_B, _D), _F32),
            jax.ShapeDtypeStruct((_B, _D), _F32),
        ],
        scratch_types=[
            pltpu.VMEM((_BPW,), jnp.int32),
            pltpu.VMEM((_BPW, _D), _F32),
            pltpu.SemaphoreType.DMA,
        ],
        compiler_params=pltpu.CompilerParams(use_tc_tiling_on_sc=False),
    )
    def k(g_tab_hbm, i_tab_hbm, gidx_hbm, iidx_hbm, gout_hbm, iout_hbm,
          idx_v, rows_v, sem):
        wid = lax.axis_index("s") * _NC + lax.axis_index("c")
        base = wid * _BPW
        pltpu.sync_copy(gidx_hbm.at[pl.ds(base, _BPW)], idx_v)
        pltpu.async_copy(g_tab_hbm.at[idx_v], rows_v, sem).wait()
        pltpu.sync_copy(rows_v, gout_hbm.at[pl.ds(base, _BPW)])
        pltpu.sync_copy(iidx_hbm.at[pl.ds(base, _BPW)], idx_v)
        pltpu.async_copy(i_tab_hbm.at[idx_v], rows_v, sem).wait()
        pltpu.sync_copy(rows_v, iout_hbm.at[pl.ds(base, _BPW)])

    return k(g_tab, i_tab, g_idx, i_idx)


def kernel(user_table, item_table, group_table, user_hyper, item_hyper,
           full_hyper, overlap_graph, lgcn_graph, W_agg, b_agg,
           W_ov, b_ov, W_hy, b_hy, W_lg, b_lg,
           group_inputs, item_inputs):
    b_agg3 = b_agg.reshape(2, 1, _D)
    group_emb = _overlap_conv(overlap_graph, group_table)
    i_emb_full, group_ui_emb = _fused_dense(
        user_table, item_table, group_table, group_emb, user_hyper,
        item_hyper, full_hyper, lgcn_graph, W_agg, b_agg3,
        W_ov, b_ov, W_hy, b_hy, W_lg, b_lg)
    g_sel, i_sel = _sc_gather_pair(group_ui_emb, i_emb_full,
                                   group_inputs, item_inputs)
    return _pair_dot(g_sel, i_sel)


# speedup vs baseline: 1.1380x; 1.1380x over previous
import functools

import jax
import jax.numpy as jnp
from jax import lax
from jax.experimental import pallas as pl
from jax.experimental.pallas import tpu as pltpu
from jax.experimental.pallas import tpu_sc as plsc

_U = 10000
_I = 5000
_G = 2000
_D = 64
_LG_ITEM = 3000
_N_LG = _G + _LG_ITEM
_B = 16384
_F32 = jnp.float32
_BF16 = jnp.bfloat16

_BM = 200


def _bdot(a, b):
    return jnp.dot(a.astype(_BF16), b.astype(_BF16),
                   preferred_element_type=_F32)


def _msg_from(uh_ref, ih_ref, w_ref, b_ref, ge_rows, u_st, it_st):
    um = _bdot(uh_ref[...], u_st)
    im = _bdot(ih_ref[...], it_st)
    ige = im * ge_rows
    w = w_ref[0]
    return (jnp.dot(um, w[0:_D], preferred_element_type=_F32)
            + jnp.dot(im, w[_D:2 * _D], preferred_element_type=_F32)
            + jnp.dot(ige, w[2 * _D:3 * _D], preferred_element_type=_F32)
            + b_ref[0])


def _k1_body(ov_ref, lg_ref, uh_ref, ih_ref, u_ref, it_ref, g_ref,
             w_ref, b_ref, ge_ref, lgem_ref, msg0_ref,
             c1_s, ge_s, e0_s, cur1_s):
    i = pl.program_id(0)

    @pl.when(i == 0)
    def _():
        e0_s[0:_G, :] = g_ref[...].astype(_BF16)
        e0_s[_G:, :] = it_ref[0:_LG_ITEM, :].astype(_BF16)

    @pl.when(i < 10)
    def _():
        c1_s[pl.ds(i * _BM, _BM), :] = _bdot(ov_ref[...], g_ref[...])

    @pl.when((i >= 10) & (i < 20))
    def _():
        j = i - 10
        sl = pl.ds(j * _BM, _BM)
        ge = g_ref[sl, :] + c1_s[sl, :] + _bdot(ov_ref[...], c1_s[...])
        ge_s[sl, :] = ge
        ge_ref[...] = ge

    @pl.when((i >= 20) & (i < 45))
    def _():
        j = i - 20
        cur1_s[pl.ds(j * _BM, _BM), :] = _bdot(lg_ref[...], e0_s[...])

    @pl.when((i >= 45) & (i < 55))
    def _():
        j = i - 45
        sl = pl.ds(j * _BM, _BM)
        lgem_ref[...] = (g_ref[sl, :] + cur1_s[sl, :]
                         + _bdot(lg_ref[...], cur1_s[...])) * (1.0 / 3.0)

    @pl.when(i >= 55)
    def _():
        m = i - 55
        msg0_ref[...] = _msg_from(uh_ref, ih_ref, w_ref, b_ref,
                                  ge_s[pl.ds(m * _BM, _BM), :],
                                  u_ref[...], it_ref[...])


def _k1(overlap_graph, lgcn_graph, user_hyper, item_hyper,
        user_table, item_table, group_table, W_agg, b_agg3):
    c2 = lambda i: (0, 0)
    return pl.pallas_call(
        _k1_body,
        grid=(65,),
        in_specs=[
            pl.BlockSpec((_BM, _G),
                         lambda i: (jnp.where(i < 10, i,
                                              jnp.clip(i - 10, 0, 9)), 0)),
            pl.BlockSpec((_BM, _N_LG),
                         lambda i: (jnp.where(
                             i < 20, 0,
                             jnp.where(i < 45, i - 20,
                                       jnp.clip(i - 45, 0, 9))), 0)),
            pl.BlockSpec((_BM, _U),
                         lambda i: (jnp.clip(i - 55, 0, 9), 0)),
            pl.BlockSpec((_BM, _I),
                         lambda i: (jnp.clip(i - 55, 0, 9), 0)),
            pl.BlockSpec((_U, _D), c2),
            pl.BlockSpec((_I, _D), c2),
            pl.BlockSpec((_G, _D), c2),
            pl.BlockSpec((1, 3 * _D, _D), lambda i: (0, 0, 0)),
            pl.BlockSpec((1, 1, _D), lambda i: (0, 0, 0)),
        ],
        out_specs=[
            pl.BlockSpec((_BM, _D), lambda i: (jnp.clip(i - 10, 0, 9), 0)),
            pl.BlockSpec((_BM, _D), lambda i: (jnp.clip(i - 45, 0, 9), 0)),
            pl.BlockSpec((_BM, _D), lambda i: (jnp.clip(i - 55, 0, 9), 0)),
        ],
        out_shape=[
            jax.ShapeDtypeStruct((_G, _D), _F32),
            jax.ShapeDtypeStruct((_G, _D), _F32),
            jax.ShapeDtypeStruct((_G, _D), _F32),
        ],
        scratch_shapes=[pltpu.VMEM((_G, _D), _F32),
                        pltpu.VMEM((_G, _D), _F32),
                        pltpu.VMEM((_N_LG, _D), _BF16),
                        pltpu.VMEM((_N_LG, _D), _F32)],
    )(overlap_graph, lgcn_graph, user_hyper, item_hyper,
      user_table, item_table, group_table, W_agg, b_agg3)


_FH_BM = 1000


def _k2_body(fh_ref, uh_ref, ih_ref, msg0_ref, ge_ref, w_ref, b_ref,
             n0i_ref, msg1_ref, norm0_s):
    i = pl.program_id(0)

    @pl.when(i < 15)
    def _():
        blk = _bdot(fh_ref[...], msg0_ref[...])
        norm0_s[pl.ds(i * _FH_BM, _FH_BM), :] = blk

        @pl.when(i >= 10)
        def _():
            n0i_ref[...] = blk

    @pl.when(i >= 15)
    def _():
        m = i - 15
        msg1_ref[...] = _msg_from(uh_ref, ih_ref, w_ref, b_ref,
                                  ge_ref[pl.ds(m * _BM, _BM), :],
                                  norm0_s[0:_U, :], norm0_s[_U:, :])


def _k2(full_hyper, user_hyper, item_hyper, msg0, group_emb, W_agg, b_agg3):
    c2 = lambda i: (0, 0)
    return pl.pallas_call(
        _k2_body,
        grid=(25,),
        in_specs=[
            pl.BlockSpec((_FH_BM, _G), lambda i: (jnp.clip(i, 0, 14), 0)),
            pl.BlockSpec((_BM, _U), lambda i: (jnp.clip(i - 15, 0, 9), 0)),
            pl.BlockSpec((_BM, _I), lambda i: (jnp.clip(i - 15, 0, 9), 0)),
            pl.BlockSpec((_G, _D), c2),
            pl.BlockSpec((_G, _D), c2),
            pl.BlockSpec((1, 3 * _D, _D), lambda i: (1, 0, 0)),
            pl.BlockSpec((1, 1, _D), lambda i: (1, 0, 0)),
        ],
        out_specs=[
            pl.BlockSpec((_FH_BM, _D), lambda i: (jnp.clip(i - 10, 0, 4), 0)),
            pl.BlockSpec((_BM, _D), lambda i: (jnp.clip(i - 15, 0, 9), 0)),
        ],
        out_shape=[
            jax.ShapeDtypeStruct((_I, _D), _F32),
            jax.ShapeDtypeStruct((_G, _D), _F32),
        ],
        scratch_shapes=[pltpu.VMEM((_U + _I, _D), _F32)],
    )(full_hyper, user_hyper, item_hyper, msg0, group_emb, W_agg, b_agg3)


def _k3_body(fh_ref, msg1_ref, it_ref, n0i_ref, ge_ref, m0_ref, lgem_ref,
             wov_ref, bov_ref, why_ref, bhy_ref, wlg_ref, blg_ref,
             iemb_ref, gui_ref):
    msg = msg1_ref[...]
    iemb_ref[...] = (it_ref[...] + n0i_ref[...] + _bdot(fh_ref[...], msg))

    @pl.when(pl.program_id(0) == 0)
    def _():
        ge = ge_ref[...]
        he = ge + m0_ref[...] + msg
        lg = lgem_ref[...]
        co = jax.nn.sigmoid(jnp.dot(ge, wov_ref[...],
                                    preferred_element_type=_F32) + bov_ref[...])
        ch = jax.nn.sigmoid(jnp.dot(he, why_ref[...],
                                    preferred_element_type=_F32) + bhy_ref[...])
        cl = jax.nn.sigmoid(jnp.dot(lg, wlg_ref[...],
                                    preferred_element_type=_F32) + blg_ref[...])
        gui_ref[...] = co * ge + ch * he + cl * lg


def _k3(full_hyper, msg1, item_table, n0_items, group_emb, msg0, lg_emb,
        wov, bov, why, bhy, wlg, blg):
    c2 = lambda i: (0, 0)
    off = _U // _FH_BM
    return pl.pallas_call(
        _k3_body,
        grid=(_I // _FH_BM,),
        in_specs=[
            pl.BlockSpec((_FH_BM, _G), lambda i: (i + off, 0)),
            pl.BlockSpec((_G, _D), c2),
            pl.BlockSpec((_FH_BM, _D), lambda i: (i, 0)),
            pl.BlockSpec((_FH_BM, _D), lambda i: (i, 0)),
            pl.BlockSpec((_G, _D), c2),
            pl.BlockSpec((_G, _D), c2),
            pl.BlockSpec((_G, _D), c2),
            pl.BlockSpec((_D, 1), c2),
            pl.BlockSpec((1,), lambda i: (0,)),
            pl.BlockSpec((_D, 1), c2),
            pl.BlockSpec((1,), lambda i: (0,)),
            pl.BlockSpec((_D, 1), c2),
            pl.BlockSpec((1,), lambda i: (0,)),
        ],
        out_specs=[
            pl.BlockSpec((_FH_BM, _D), lambda i: (i, 0)),
            pl.BlockSpec((_G, _D), lambda i: (0, 0)),
        ],
        out_shape=[
            jax.ShapeDtypeStruct((_I, _D), _F32),
            jax.ShapeDtypeStruct((_G, _D), _F32),
        ],
    )(full_hyper, msg1, item_table, n0_items, group_emb, msg0, lg_emb,
      wov, bov, why, bhy, wlg, blg)


def _dot_body(g_ref, i_ref, out_ref):
    out_ref[...] = jnp.sum(g_ref[...] * i_ref[...], axis=1)


def _pair_dot(g_sel, i_sel):
    bm = 4096
    return pl.pallas_call(
        _dot_body,
        grid=(_B // bm,),
        in_specs=[
            pl.BlockSpec((bm, _D), lambda i: (i, 0)),
            pl.BlockSpec((bm, _D), lambda i: (i, 0)),
        ],
        out_specs=pl.BlockSpec((bm,), lambda i: (i,)),
        out_shape=jax.ShapeDtypeStruct((_B,), _F32),
    )(g_sel, i_sel)


_NC = 2
_NS = 16
_NW = _NC * _NS
_BPW = _B // _NW


def _sc_gather_pair(g_tab, i_tab, g_idx, i_idx):
    mesh = plsc.VectorSubcoreMesh(core_axis_name="c", subcore_axis_name="s")

    @functools.partial(
        pl.kernel,
        mesh=mesh,
        out_type=[
            jax.ShapeDtypeStruct((_B, _D), _F32),
            jax.ShapeDtypeStruct((_B, _D), _F32),
        ],
        scratch_types=[
            pltpu.VMEM((_BPW,), jnp.int32),
            pltpu.VMEM((_BPW, _D), _F32),
            pltpu.SemaphoreType.DMA,
        ],
        compiler_params=pltpu.CompilerParams(use_tc_tiling_on_sc=False),
    )
    def k(g_tab_hbm, i_tab_hbm, gidx_hbm, iidx_hbm, gout_hbm, iout_hbm,
          idx_v, rows_v, sem):
        wid = lax.axis_index("s") * _NC + lax.axis_index("c")
        base = wid * _BPW
        pltpu.sync_copy(gidx_hbm.at[pl.ds(base, _BPW)], idx_v)
        pltpu.async_copy(g_tab_hbm.at[idx_v], rows_v, sem).wait()
        pltpu.sync_copy(rows_v, gout_hbm.at[pl.ds(base, _BPW)])
        pltpu.sync_copy(iidx_hbm.at[pl.ds(base, _BPW)], idx_v)
        pltpu.async_copy(i_tab_hbm.at[idx_v], rows_v, sem).wait()
        pltpu.sync_copy(rows_v, iout_hbm.at[pl.ds(base, _BPW)])

    return k(g_tab, i_tab, g_idx, i_idx)


def kernel(user_table, item_table, group_table, user_hyper, item_hyper,
           full_hyper, overlap_graph, lgcn_graph, W_agg, b_agg,
           W_ov, b_ov, W_hy, b_hy, W_lg, b_lg,
           group_inputs, item_inputs):
    b_agg3 = b_agg.reshape(2, 1, _D)
    group_emb, lg_emb, msg0 = _k1(
        overlap_graph, lgcn_graph, user_hyper, item_hyper,
        user_table, item_table, group_table, W_agg, b_agg3)
    n0_items, msg1 = _k2(full_hyper, user_hyper, item_hyper, msg0,
                         group_emb, W_agg, b_agg3)
    i_emb_full, group_ui_emb = _k3(
        full_hyper, msg1, item_table, n0_items, group_emb, msg0, lg_emb,
        W_ov, b_ov, W_hy, b_hy, W_lg, b_lg)
    g_sel, i_sel = _sc_gather_pair(group_ui_emb, i_emb_full,
                                   group_inputs, item_inputs)
    return _pair_dot(g_sel, i_sel)


# speedup vs baseline: 1.1594x; 1.0189x over previous
import functools

import jax
import jax.numpy as jnp
from jax import lax
from jax.experimental import pallas as pl
from jax.experimental.pallas import tpu as pltpu
from jax.experimental.pallas import tpu_sc as plsc

_U = 10000
_I = 5000
_G = 2000
_D = 64
_LG_ITEM = 3000
_N_LG = _G + _LG_ITEM
_B = 16384
_F32 = jnp.float32
_BF16 = jnp.bfloat16

_BM = 200


def _bdot(a, b):
    return jnp.dot(a.astype(_BF16), b.astype(_BF16),
                   preferred_element_type=_F32)


def _msg_from(uh_ref, ih_ref, w_ref, b_ref, ge_rows, u_st, it_st):
    um = _bdot(uh_ref[...], u_st)
    im = _bdot(ih_ref[...], it_st)
    ige = im * ge_rows
    w = w_ref[0]
    return (jnp.dot(um, w[0:_D], preferred_element_type=_F32)
            + jnp.dot(im, w[_D:2 * _D], preferred_element_type=_F32)
            + jnp.dot(ige, w[2 * _D:3 * _D], preferred_element_type=_F32)
            + b_ref[0])


_OV_SPLIT = 5


def _overlap_body(*args):
    a_parts, (g_ref, out_ref, c1_ref) = args[:_OV_SPLIT], args[_OV_SPLIT:]
    q = _G // _OV_SPLIT
    g = g_ref[...]
    for k, a_ref in enumerate(a_parts):
        c1_ref[pl.ds(k * q, q), :] = _bdot(a_ref[...], g)
    c1 = c1_ref[...]
    for k, a_ref in enumerate(a_parts):
        sl = pl.ds(k * q, q)
        out_ref[sl, :] = g_ref[sl, :] + c1_ref[sl, :] + _bdot(a_ref[...], c1)


def _overlap_conv(overlap_graph, group_table):
    q = _G // _OV_SPLIT
    return pl.pallas_call(
        _overlap_body,
        grid=(1,),
        in_specs=[pl.BlockSpec((q, _G), lambda i, k=k: (k, 0))
                  for k in range(_OV_SPLIT)]
        + [pl.BlockSpec((_G, _D), lambda i: (0, 0))],
        out_specs=pl.BlockSpec((_G, _D), lambda i: (0, 0)),
        out_shape=jax.ShapeDtypeStruct((_G, _D), _F32),
        scratch_shapes=[pltpu.VMEM((_G, _D), _F32)],
    )(*([overlap_graph] * _OV_SPLIT + [group_table]))


def _k1_body(lg_ref, uh_ref, ih_ref, u_ref, it_ref, g_ref, ge_ref,
             w_ref, b_ref, lgem_ref, msg0_ref,
             e0_s, cur1_s):
    i = pl.program_id(0)

    @pl.when(i == 0)
    def _():
        e0_s[0:_G, :] = g_ref[...].astype(_BF16)
        e0_s[_G:, :] = it_ref[0:_LG_ITEM, :].astype(_BF16)

    @pl.when(i < 25)
    def _():
        cur1_s[pl.ds(i * _BM, _BM), :] = _bdot(lg_ref[...], e0_s[...])

    @pl.when((i >= 25) & (i < 35))
    def _():
        j = i - 25
        sl = pl.ds(j * _BM, _BM)
        lgem_ref[...] = (g_ref[sl, :] + cur1_s[sl, :]
                         + _bdot(lg_ref[...], cur1_s[...])) * (1.0 / 3.0)

    @pl.when(i >= 35)
    def _():
        m = i - 35
        msg0_ref[...] = _msg_from(uh_ref, ih_ref, w_ref, b_ref,
                                  ge_ref[pl.ds(m * _BM, _BM), :],
                                  u_ref[...], it_ref[...])


def _k1(lgcn_graph, user_hyper, item_hyper,
        user_table, item_table, group_table, group_emb, W_agg, b_agg3):
    c2 = lambda i: (0, 0)
    return pl.pallas_call(
        _k1_body,
        grid=(45,),
        in_specs=[
            pl.BlockSpec((_BM, _N_LG),
                         lambda i: (jnp.where(i < 25, i,
                                              jnp.clip(i - 25, 0, 9)), 0)),
            pl.BlockSpec((_BM, _U),
                         lambda i: (jnp.clip(i - 35, 0, 9), 0)),
            pl.BlockSpec((_BM, _I),
                         lambda i: (jnp.clip(i - 35, 0, 9), 0)),
            pl.BlockSpec((_U, _D), c2),
            pl.BlockSpec((_I, _D), c2),
            pl.BlockSpec((_G, _D), c2),
            pl.BlockSpec((_G, _D), c2),
            pl.BlockSpec((1, 3 * _D, _D), lambda i: (0, 0, 0)),
            pl.BlockSpec((1, 1, _D), lambda i: (0, 0, 0)),
        ],
        out_specs=[
            pl.BlockSpec((_BM, _D), lambda i: (jnp.clip(i - 25, 0, 9), 0)),
            pl.BlockSpec((_BM, _D), lambda i: (jnp.clip(i - 35, 0, 9), 0)),
        ],
        out_shape=[
            jax.ShapeDtypeStruct((_G, _D), _F32),
            jax.ShapeDtypeStruct((_G, _D), _F32),
        ],
        scratch_shapes=[pltpu.VMEM((_N_LG, _D), _BF16),
                        pltpu.VMEM((_N_LG, _D), _F32)],
    )(lgcn_graph, user_hyper, item_hyper,
      user_table, item_table, group_table, group_emb, W_agg, b_agg3)


_FH_BM = 1000


def _k2_body(fh_ref, uh_ref, ih_ref, msg0_ref, ge_ref, w_ref, b_ref,
             n0i_ref, msg1_ref, norm0_s):
    i = pl.program_id(0)

    @pl.when(i < 15)
    def _():
        blk = _bdot(fh_ref[...], msg0_ref[...])
        norm0_s[pl.ds(i * _FH_BM, _FH_BM), :] = blk

        @pl.when(i >= 10)
        def _():
            n0i_ref[...] = blk

    @pl.when(i >= 15)
    def _():
        m = i - 15
        msg1_ref[...] = _msg_from(uh_ref, ih_ref, w_ref, b_ref,
                                  ge_ref[pl.ds(m * _BM, _BM), :],
                                  norm0_s[0:_U, :], norm0_s[_U:, :])


def _k2(full_hyper, user_hyper, item_hyper, msg0, group_emb, W_agg, b_agg3):
    c2 = lambda i: (0, 0)
    return pl.pallas_call(
        _k2_body,
        grid=(25,),
        in_specs=[
            pl.BlockSpec((_FH_BM, _G), lambda i: (jnp.clip(i, 0, 14), 0)),
            pl.BlockSpec((_BM, _U), lambda i: (jnp.clip(i - 15, 0, 9), 0)),
            pl.BlockSpec((_BM, _I), lambda i: (jnp.clip(i - 15, 0, 9), 0)),
            pl.BlockSpec((_G, _D), c2),
            pl.BlockSpec((_G, _D), c2),
            pl.BlockSpec((1, 3 * _D, _D), lambda i: (1, 0, 0)),
            pl.BlockSpec((1, 1, _D), lambda i: (1, 0, 0)),
        ],
        out_specs=[
            pl.BlockSpec((_FH_BM, _D), lambda i: (jnp.clip(i - 10, 0, 4), 0)),
            pl.BlockSpec((_BM, _D), lambda i: (jnp.clip(i - 15, 0, 9), 0)),
        ],
        out_shape=[
            jax.ShapeDtypeStruct((_I, _D), _F32),
            jax.ShapeDtypeStruct((_G, _D), _F32),
        ],
        scratch_shapes=[pltpu.VMEM((_U + _I, _D), _F32)],
    )(full_hyper, user_hyper, item_hyper, msg0, group_emb, W_agg, b_agg3)


def _k3_body(fh_ref, msg1_ref, it_ref, n0i_ref, ge_ref, m0_ref, lgem_ref,
             wov_ref, bov_ref, why_ref, bhy_ref, wlg_ref, blg_ref,
             iemb_ref, gui_ref):
    msg = msg1_ref[...]
    iemb_ref[...] = (it_ref[...] + n0i_ref[...] + _bdot(fh_ref[...], msg))

    @pl.when(pl.program_id(0) == 0)
    def _():
        ge = ge_ref[...]
        he = ge + m0_ref[...] + msg
        lg = lgem_ref[...]
        co = jax.nn.sigmoid(jnp.dot(ge, wov_ref[...],
                                    preferred_element_type=_F32) + bov_ref[...])
        ch = jax.nn.sigmoid(jnp.dot(he, why_ref[...],
                                    preferred_element_type=_F32) + bhy_ref[...])
        cl = jax.nn.sigmoid(jnp.dot(lg, wlg_ref[...],
                                    preferred_element_type=_F32) + blg_ref[...])
        gui_ref[...] = co * ge + ch * he + cl * lg


def _k3(full_hyper, msg1, item_table, n0_items, group_emb, msg0, lg_emb,
        wov, bov, why, bhy, wlg, blg):
    c2 = lambda i: (0, 0)
    off = _U // _FH_BM
    return pl.pallas_call(
        _k3_body,
        grid=(_I // _FH_BM,),
        in_specs=[
            pl.BlockSpec((_FH_BM, _G), lambda i: (i + off, 0)),
            pl.BlockSpec((_G, _D), c2),
            pl.BlockSpec((_FH_BM, _D), lambda i: (i, 0)),
            pl.BlockSpec((_FH_BM, _D), lambda i: (i, 0)),
            pl.BlockSpec((_G, _D), c2),
            pl.BlockSpec((_G, _D), c2),
            pl.BlockSpec((_G, _D), c2),
            pl.BlockSpec((_D, 1), c2),
            pl.BlockSpec((1,), lambda i: (0,)),
            pl.BlockSpec((_D, 1), c2),
            pl.BlockSpec((1,), lambda i: (0,)),
            pl.BlockSpec((_D, 1), c2),
            pl.BlockSpec((1,), lambda i: (0,)),
        ],
        out_specs=[
            pl.BlockSpec((_FH_BM, _D), lambda i: (i, 0)),
            pl.BlockSpec((_G, _D), lambda i: (0, 0)),
        ],
        out_shape=[
            jax.ShapeDtypeStruct((_I, _D), _F32),
            jax.ShapeDtypeStruct((_G, _D), _F32),
        ],
    )(full_hyper, msg1, item_table, n0_items, group_emb, msg0, lg_emb,
      wov, bov, why, bhy, wlg, blg)


def _dot_body(g_ref, i_ref, out_ref):
    out_ref[...] = jnp.sum(g_ref[...] * i_ref[...], axis=1)


def _pair_dot(g_sel, i_sel):
    bm = 4096
    return pl.pallas_call(
        _dot_body,
        grid=(_B // bm,),
        in_specs=[
            pl.BlockSpec((bm, _D), lambda i: (i, 0)),
            pl.BlockSpec((bm, _D), lambda i: (i, 0)),
        ],
        out_specs=pl.BlockSpec((bm,), lambda i: (i,)),
        out_shape=jax.ShapeDtypeStruct((_B,), _F32),
    )(g_sel, i_sel)


_NC = 2
_NS = 16
_NW = _NC * _NS
_BPW = _B // _NW


def _sc_gather_pair(g_tab, i_tab, g_idx, i_idx):
    mesh = plsc.VectorSubcoreMesh(core_axis_name="c", subcore_axis_name="s")

    @functools.partial(
        pl.kernel,
        mesh=mesh,
        out_type=[
            jax.ShapeDtypeStruct((_B, _D), _F32),
            jax.ShapeDtypeStruct((_B, _D), _F32),
        ],
        scratch_types=[
            pltpu.VMEM((_BPW,), jnp.int32),
            pltpu.VMEM((_BPW,), jnp.int32),
            pltpu.VMEM((_BPW, _D), _F32),
            pltpu.VMEM((_BPW, _D), _F32),
            pltpu.SemaphoreType.DMA,
            pltpu.SemaphoreType.DMA,
        ],
        compiler_params=pltpu.CompilerParams(use_tc_tiling_on_sc=False),
    )
    def k(g_tab_hbm, i_tab_hbm, gidx_hbm, iidx_hbm, gout_hbm, iout_hbm,
          gidx_v, iidx_v, grows_v, irows_v, gsem, isem):
        wid = lax.axis_index("s") * _NC + lax.axis_index("c")
        base = wid * _BPW
        pltpu.sync_copy(gidx_hbm.at[pl.ds(base, _BPW)], gidx_v)
        pltpu.sync_copy(iidx_hbm.at[pl.ds(base, _BPW)], iidx_v)
        gcp = pltpu.async_copy(g_tab_hbm.at[gidx_v], grows_v, gsem)
        icp = pltpu.async_copy(i_tab_hbm.at[iidx_v], irows_v, isem)
        gcp.wait()
        icp.wait()
        pltpu.sync_copy(grows_v, gout_hbm.at[pl.ds(base, _BPW)])
        pltpu.sync_copy(irows_v, iout_hbm.at[pl.ds(base, _BPW)])

    return k(g_tab, i_tab, g_idx, i_idx)


def kernel(user_table, item_table, group_table, user_hyper, item_hyper,
           full_hyper, overlap_graph, lgcn_graph, W_agg, b_agg,
           W_ov, b_ov, W_hy, b_hy, W_lg, b_lg,
           group_inputs, item_inputs):
    b_agg3 = b_agg.reshape(2, 1, _D)
    group_emb = _overlap_conv(overlap_graph, group_table)
    lg_emb, msg0 = _k1(
        lgcn_graph, user_hyper, item_hyper,
        user_table, item_table, group_table, group_emb, W_agg, b_agg3)
    n0_items, msg1 = _k2(full_hyper, user_hyper, item_hyper, msg0,
                         group_emb, W_agg, b_agg3)
    i_emb_full, group_ui_emb = _k3(
        full_hyper, msg1, item_table, n0_items, group_emb, msg0, lg_emb,
        W_ov, b_ov, W_hy, b_hy, W_lg, b_lg)
    g_sel, i_sel = _sc_gather_pair(group_ui_emb, i_emb_full,
                                   group_inputs, item_inputs)
    return _pair_dot(g_sel, i_sel)


# speedup vs baseline: 1.1661x; 1.0057x over previous
import functools

import jax
import jax.numpy as jnp
from jax import lax
from jax.experimental import pallas as pl
from jax.experimental.pallas import tpu as pltpu
from jax.experimental.pallas import tpu_sc as plsc

_U = 10000
_I = 5000
_G = 2000
_D = 64
_LG_ITEM = 3000
_N_LG = _G + _LG_ITEM
_B = 16384
_F32 = jnp.float32
_BF16 = jnp.bfloat16

_BM = 200


def _bdot(a, b):
    return jnp.dot(a.astype(_BF16), b.astype(_BF16),
                   preferred_element_type=_F32)


def _msg_from(uh_ref, ih_ref, w_ref, b_ref, ge_rows, u_st, it_st):
    um = _bdot(uh_ref[...], u_st)
    im = _bdot(ih_ref[...], it_st)
    ige = im * ge_rows
    w = w_ref[0]
    return (jnp.dot(um, w[0:_D], preferred_element_type=_F32)
            + jnp.dot(im, w[_D:2 * _D], preferred_element_type=_F32)
            + jnp.dot(ige, w[2 * _D:3 * _D], preferred_element_type=_F32)
            + b_ref[0])


_OV_SPLIT = 5


def _k1_body(*args):
    a_parts = args[:_OV_SPLIT]
    (lg_ref, uh_ref, ih_ref, u_ref, it_ref, g_ref,
     w_ref, b_ref, ge_ref, lgem_ref, msg0_ref,
     c1_s, ge_s, e0_s, cur1_s) = args[_OV_SPLIT:]
    i = pl.program_id(0)

    @pl.when(i == 0)
    def _():
        e0_s[0:_G, :] = g_ref[...].astype(_BF16)
        e0_s[_G:, :] = it_ref[0:_LG_ITEM, :].astype(_BF16)
        q = _G // _OV_SPLIT
        g = g_ref[...]
        for k, a_ref in enumerate(a_parts):
            c1_s[pl.ds(k * q, q), :] = _bdot(a_ref[...], g)
        c1 = c1_s[...]
        for k, a_ref in enumerate(a_parts):
            sl = pl.ds(k * q, q)
            ge_s[sl, :] = g_ref[sl, :] + c1_s[sl, :] + _bdot(a_ref[...], c1)
        ge_ref[...] = ge_s[...]

    @pl.when((i >= 1) & (i < 26))
    def _():
        j = i - 1
        cur1_s[pl.ds(j * _BM, _BM), :] = _bdot(lg_ref[...], e0_s[...])

    @pl.when((i >= 26) & (i < 36))
    def _():
        j = i - 26
        sl = pl.ds(j * _BM, _BM)
        lgem_ref[...] = (g_ref[sl, :] + cur1_s[sl, :]
                         + _bdot(lg_ref[...], cur1_s[...])) * (1.0 / 3.0)

    @pl.when(i >= 36)
    def _():
        m = i - 36
        msg0_ref[...] = _msg_from(uh_ref, ih_ref, w_ref, b_ref,
                                  ge_s[pl.ds(m * _BM, _BM), :],
                                  u_ref[...], it_ref[...])


def _k1(overlap_graph, lgcn_graph, user_hyper, item_hyper,
        user_table, item_table, group_table, W_agg, b_agg3):
    c2 = lambda i: (0, 0)
    q = _G // _OV_SPLIT
    return pl.pallas_call(
        _k1_body,
        grid=(46,),
        in_specs=[pl.BlockSpec((q, _G), lambda i, k=k: (k, 0))
                  for k in range(_OV_SPLIT)]
        + [
            pl.BlockSpec((_BM, _N_LG),
                         lambda i: (jnp.where(i < 26, jnp.maximum(i - 1, 0),
                                              jnp.clip(i - 26, 0, 9)), 0)),
            pl.BlockSpec((_BM, _U),
                         lambda i: (jnp.clip(i - 36, 0, 9), 0)),
            pl.BlockSpec((_BM, _I),
                         lambda i: (jnp.clip(i - 36, 0, 9), 0)),
            pl.BlockSpec((_U, _D), c2),
            pl.BlockSpec((_I, _D), c2),
            pl.BlockSpec((_G, _D), c2),
            pl.BlockSpec((1, 3 * _D, _D), lambda i: (0, 0, 0)),
            pl.BlockSpec((1, 1, _D), lambda i: (0, 0, 0)),
        ],
        out_specs=[
            pl.BlockSpec((_G, _D), lambda i: (0, 0)),
            pl.BlockSpec((_BM, _D), lambda i: (jnp.clip(i - 26, 0, 9), 0)),
            pl.BlockSpec((_BM, _D), lambda i: (jnp.clip(i - 36, 0, 9), 0)),
        ],
        out_shape=[
            jax.ShapeDtypeStruct((_G, _D), _F32),
            jax.ShapeDtypeStruct((_G, _D), _F32),
            jax.ShapeDtypeStruct((_G, _D), _F32),
        ],
        scratch_shapes=[pltpu.VMEM((_G, _D), _F32),
                        pltpu.VMEM((_G, _D), _F32),
                        pltpu.VMEM((_N_LG, _D), _BF16),
                        pltpu.VMEM((_N_LG, _D), _F32)],
        compiler_params=pltpu.CompilerParams(
            vmem_limit_bytes=67 * 1024 * 1024),
    )(*([overlap_graph] * _OV_SPLIT
        + [lgcn_graph, user_hyper, item_hyper,
           user_table, item_table, group_table, W_agg, b_agg3]))


_FH_BM = 1000


def _k2_body(fh_ref, uh_ref, ih_ref, msg0_ref, ge_ref, w_ref, b_ref,
             n0i_ref, msg1_ref, norm0_s):
    i = pl.program_id(0)

    @pl.when(i < 15)
    def _():
        blk = _bdot(fh_ref[...], msg0_ref[...])
        norm0_s[pl.ds(i * _FH_BM, _FH_BM), :] = blk

        @pl.when(i >= 10)
        def _():
            n0i_ref[...] = blk

    @pl.when(i >= 15)
    def _():
        m = i - 15
        msg1_ref[...] = _msg_from(uh_ref, ih_ref, w_ref, b_ref,
                                  ge_ref[pl.ds(m * _BM, _BM), :],
                                  norm0_s[0:_U, :], norm0_s[_U:, :])


def _k2(full_hyper, user_hyper, item_hyper, msg0, group_emb, W_agg, b_agg3):
    c2 = lambda i: (0, 0)
    return pl.pallas_call(
        _k2_body,
        grid=(25,),
        in_specs=[
            pl.BlockSpec((_FH_BM, _G), lambda i: (jnp.clip(i, 0, 14), 0)),
            pl.BlockSpec((_BM, _U), lambda i: (jnp.clip(i - 15, 0, 9), 0)),
            pl.BlockSpec((_BM, _I), lambda i: (jnp.clip(i - 15, 0, 9), 0)),
            pl.BlockSpec((_G, _D), c2),
            pl.BlockSpec((_G, _D), c2),
            pl.BlockSpec((1, 3 * _D, _D), lambda i: (1, 0, 0)),
            pl.BlockSpec((1, 1, _D), lambda i: (1, 0, 0)),
        ],
        out_specs=[
            pl.BlockSpec((_FH_BM, _D), lambda i: (jnp.clip(i - 10, 0, 4), 0)),
            pl.BlockSpec((_BM, _D), lambda i: (jnp.clip(i - 15, 0, 9), 0)),
        ],
        out_shape=[
            jax.ShapeDtypeStruct((_I, _D), _F32),
            jax.ShapeDtypeStruct((_G, _D), _F32),
        ],
        scratch_shapes=[pltpu.VMEM((_U + _I, _D), _F32)],
    )(full_hyper, user_hyper, item_hyper, msg0, group_emb, W_agg, b_agg3)


def _k3_body(fh_ref, msg1_ref, it_ref, n0i_ref, ge_ref, m0_ref, lgem_ref,
             wov_ref, bov_ref, why_ref, bhy_ref, wlg_ref, blg_ref,
             iemb_ref, gui_ref):
    msg = msg1_ref[...]
    iemb_ref[...] = (it_ref[...] + n0i_ref[...] + _bdot(fh_ref[...], msg))

    @pl.when(pl.program_id(0) == 0)
    def _():
        ge = ge_ref[...]
        he = ge + m0_ref[...] + msg
        lg = lgem_ref[...]
        co = jax.nn.sigmoid(jnp.dot(ge, wov_ref[...],
                                    preferred_element_type=_F32) + bov_ref[...])
        ch = jax.nn.sigmoid(jnp.dot(he, why_ref[...],
                                    preferred_element_type=_F32) + bhy_ref[...])
        cl = jax.nn.sigmoid(jnp.dot(lg, wlg_ref[...],
                                    preferred_element_type=_F32) + blg_ref[...])
        gui_ref[...] = co * ge + ch * he + cl * lg


def _k3(full_hyper, msg1, item_table, n0_items, group_emb, msg0, lg_emb,
        wov, bov, why, bhy, wlg, blg):
    c2 = lambda i: (0, 0)
    off = _U // _FH_BM
    return pl.pallas_call(
        _k3_body,
        grid=(_I // _FH_BM,),
        in_specs=[
            pl.BlockSpec((_FH_BM, _G), lambda i: (i + off, 0)),
            pl.BlockSpec((_G, _D), c2),
            pl.BlockSpec((_FH_BM, _D), lambda i: (i, 0)),
            pl.BlockSpec((_FH_BM, _D), lambda i: (i, 0)),
            pl.BlockSpec((_G, _D), c2),
            pl.BlockSpec((_G, _D), c2),
            pl.BlockSpec((_G, _D), c2),
            pl.BlockSpec((_D, 1), c2),
            pl.BlockSpec((1,), lambda i: (0,)),
            pl.BlockSpec((_D, 1), c2),
            pl.BlockSpec((1,), lambda i: (0,)),
            pl.BlockSpec((_D, 1), c2),
            pl.BlockSpec((1,), lambda i: (0,)),
        ],
        out_specs=[
            pl.BlockSpec((_FH_BM, _D), lambda i: (i, 0)),
            pl.BlockSpec((_G, _D), lambda i: (0, 0)),
        ],
        out_shape=[
            jax.ShapeDtypeStruct((_I, _D), _F32),
            jax.ShapeDtypeStruct((_G, _D), _F32),
        ],
    )(full_hyper, msg1, item_table, n0_items, group_emb, msg0, lg_emb,
      wov, bov, why, bhy, wlg, blg)


def _dot_body(g_ref, i_ref, out_ref):
    out_ref[...] = jnp.sum(g_ref[...] * i_ref[...], axis=1)


def _pair_dot(g_sel, i_sel):
    bm = 4096
    return pl.pallas_call(
        _dot_body,
        grid=(_B // bm,),
        in_specs=[
            pl.BlockSpec((bm, _D), lambda i: (i, 0)),
            pl.BlockSpec((bm, _D), lambda i: (i, 0)),
        ],
        out_specs=pl.BlockSpec((bm,), lambda i: (i,)),
        out_shape=jax.ShapeDtypeStruct((_B,), _F32),
    )(g_sel, i_sel)


_NC = 2
_NS = 16
_NW = _NC * _NS
_BPW = _B // _NW


def _sc_gather_pair(g_tab, i_tab, g_idx, i_idx):
    mesh = plsc.VectorSubcoreMesh(core_axis_name="c", subcore_axis_name="s")

    @functools.partial(
        pl.kernel,
        mesh=mesh,
        out_type=[
            jax.ShapeDtypeStruct((_B, _D), _F32),
            jax.ShapeDtypeStruct((_B, _D), _F32),
        ],
        scratch_types=[
            pltpu.VMEM((_BPW,), jnp.int32),
            pltpu.VMEM((_BPW,), jnp.int32),
            pltpu.VMEM((_BPW, _D), _F32),
            pltpu.VMEM((_BPW, _D), _F32),
            pltpu.SemaphoreType.DMA,
            pltpu.SemaphoreType.DMA,
        ],
        compiler_params=pltpu.CompilerParams(use_tc_tiling_on_sc=False),
    )
    def k(g_tab_hbm, i_tab_hbm, gidx_hbm, iidx_hbm, gout_hbm, iout_hbm,
          gidx_v, iidx_v, grows_v, irows_v, gsem, isem):
        wid = lax.axis_index("s") * _NC + lax.axis_index("c")
        base = wid * _BPW
        pltpu.sync_copy(gidx_hbm.at[pl.ds(base, _BPW)], gidx_v)
        pltpu.sync_copy(iidx_hbm.at[pl.ds(base, _BPW)], iidx_v)
        gcp = pltpu.async_copy(g_tab_hbm.at[gidx_v], grows_v, gsem)
        icp = pltpu.async_copy(i_tab_hbm.at[iidx_v], irows_v, isem)
        gcp.wait()
        icp.wait()
        pltpu.sync_copy(grows_v, gout_hbm.at[pl.ds(base, _BPW)])
        pltpu.sync_copy(irows_v, iout_hbm.at[pl.ds(base, _BPW)])

    return k(g_tab, i_tab, g_idx, i_idx)


def kernel(user_table, item_table, group_table, user_hyper, item_hyper,
           full_hyper, overlap_graph, lgcn_graph, W_agg, b_agg,
           W_ov, b_ov, W_hy, b_hy, W_lg, b_lg,
           group_inputs, item_inputs):
    b_agg3 = b_agg.reshape(2, 1, _D)
    group_emb, lg_emb, msg0 = _k1(
        overlap_graph, lgcn_graph, user_hyper, item_hyper,
        user_table, item_table, group_table, W_agg, b_agg3)
    n0_items, msg1 = _k2(full_hyper, user_hyper, item_hyper, msg0,
                         group_emb, W_agg, b_agg3)
    i_emb_full, group_ui_emb = _k3(
        full_hyper, msg1, item_table, n0_items, group_emb, msg0, lg_emb,
        W_ov, b_ov, W_hy, b_hy, W_lg, b_lg)
    g_sel, i_sel = _sc_gather_pair(group_ui_emb, i_emb_full,
                                   group_inputs, item_inputs)
    return _pair_dot(g_sel, i_sel)
